# baseline (speedup 1.0000x reference)
import functools

import jax
import jax.numpy as jnp
import numpy as np
from jax.experimental import pallas as pl
from jax.experimental.pallas import tpu as pltpu

_LEVELS = np.array([8, 8, 8, 6, 5], dtype=np.int64)
_DIM = 768
_C = len(_LEVELS)

_EPS = 1e-3
_levels_f = _LEVELS.astype(np.float32)
_half_l = ((_levels_f - 1.0) * (1.0 + _EPS) / 2.0).astype(np.float32)
_offset = np.where(_LEVELS % 2 == 0, 0.5, 0.0).astype(np.float32)
_shift = np.arctanh(_offset / _half_l).astype(np.float32)
_half_width = (_LEVELS // 2).astype(np.float32)
_basis = np.concatenate(([1], np.cumprod(_LEVELS[:-1]))).astype(np.float32)
_IDX_CONST = float(np.sum(_half_width * _basis))

_CVEC = np.zeros((8, _C), dtype=np.float32)
_CVEC[0] = _shift
_CVEC[1] = _half_l
_CVEC[2] = _offset
_CVEC[3] = _basis
_CVEC[4] = 1.0 / _half_width

_BB = 4


def _fsq_kernel(x_ref, w_in_ref, b_in_ref, cvec_ref, w_out_ref, b_out_ref,
                idx_ref, out_ref):
    bb, t, d = x_ref.shape
    x = x_ref[...].reshape(bb * t, d)
    z = jnp.dot(x, w_in_ref[...], preferred_element_type=jnp.float32)
    shift_eff = b_in_ref[...][None, :] + cvec_ref[0:1, :]
    bounded = (jnp.tanh(z + shift_eff) * cvec_ref[1:2, :]
               - cvec_ref[2:3, :])
    q = jnp.round(bounded)
    codes = q * cvec_ref[4:5, :]
    out = (jnp.dot(codes, w_out_ref[...], preferred_element_type=jnp.float32)
           + b_out_ref[...][None, :])
    out_ref[...] = out.reshape(bb, t, d)
    for k in range(bb):
        idxv = jax.lax.dot_general(cvec_ref[3:4, :], q[k * t:(k + 1) * t],
                                   (((1,), (1,)), ((), ())),
                                   preferred_element_type=jnp.float32)
        idx_ref[k] = (idxv + _IDX_CONST).astype(jnp.int32)


@functools.partial(jax.jit, static_argnames=("interpret",))
def kernel(x, W_in, b_in, W_out, b_out, interpret=False):
    B, T, D = x.shape

    idx3, out3 = pl.pallas_call(
        _fsq_kernel,
        grid=(B // _BB,),
        in_specs=[
            pl.BlockSpec((_BB, T, D), lambda i: (i, 0, 0)),
            pl.BlockSpec((D, _C), lambda i: (0, 0)),
            pl.BlockSpec((_C,), lambda i: (0,)),
            pl.BlockSpec((8, _C), lambda i: (0, 0)),
            pl.BlockSpec((_C, D), lambda i: (0, 0)),
            pl.BlockSpec((D,), lambda i: (0,)),
        ],
        out_specs=[
            pl.BlockSpec((_BB, 1, T), lambda i: (i, 0, 0)),
            pl.BlockSpec((_BB, T, D), lambda i: (i, 0, 0)),
        ],
        out_shape=[
            jax.ShapeDtypeStruct((B, 1, T), jnp.int32),
            jax.ShapeDtypeStruct((B, T, D), jnp.float32),
        ],
        compiler_params=pltpu.CompilerParams(
            dimension_semantics=("parallel",)),
        interpret=interpret,
    )(x, W_in, b_in, jnp.asarray(_CVEC), W_out, b_out)

    embed_ind = idx3.reshape(B, T)
    commit_loss = jnp.zeros((), dtype=jnp.float32)
    return (embed_ind, out3, commit_loss)


# speedup vs baseline: 1.0731x; 1.0731x over previous
import functools

import jax
import jax.numpy as jnp
import numpy as np
from jax.experimental import pallas as pl
from jax.experimental.pallas import tpu as pltpu

_LEVELS = np.array([8, 8, 8, 6, 5], dtype=np.int64)
_DIM = 768
_C = len(_LEVELS)

_EPS = 1e-3
_levels_f = _LEVELS.astype(np.float32)
_half_l = ((_levels_f - 1.0) * (1.0 + _EPS) / 2.0).astype(np.float32)
_offset = np.where(_LEVELS % 2 == 0, 0.5, 0.0).astype(np.float32)
_shift = np.arctanh(_offset / _half_l).astype(np.float32)
_half_width = (_LEVELS // 2).astype(np.float32)
_basis = np.concatenate(([1], np.cumprod(_LEVELS[:-1]))).astype(np.float32)
_IDX_CONST = float(np.sum(_half_width * _basis))

_CVECT = np.zeros((_C, 8), dtype=np.float32)
_CVECT[:, 0] = _shift
_CVECT[:, 1] = _half_l
_CVECT[:, 2] = _offset
_CVECT[:, 3] = _basis
_CVECT[:, 4] = 1.0 / _half_width

_BB = 4


def _fsq_kernel(x_ref, w_in_ref, b_in_ref, cvec_ref, w_out_ref, b_out_ref,
                idx_ref, out_ref):
    bb, t, d = x_ref.shape
    x = x_ref[...].reshape(bb * t, d)
    zt = jax.lax.dot_general(w_in_ref[...], x, (((0,), (1,)), ((), ())),
                             preferred_element_type=jnp.float32)
    shift_eff = b_in_ref[...] + cvec_ref[:, 0:1]
    bounded = (jnp.tanh(zt + shift_eff) * cvec_ref[:, 1:2]
               - cvec_ref[:, 2:3])
    qt = jnp.round(bounded)
    codes_t = qt * cvec_ref[:, 4:5]
    out = jax.lax.dot_general(codes_t, w_out_ref[...],
                              (((0,), (0,)), ((), ())),
                              preferred_element_type=jnp.float32)
    out_ref[...] = (out + b_out_ref[...][None, :]).reshape(bb, t, d)
    for k in range(bb):
        idxv = jax.lax.dot_general(cvec_ref[:, 3:4], qt[:, k * t:(k + 1) * t],
                                   (((0,), (0,)), ((), ())),
                                   preferred_element_type=jnp.float32)
        idx_ref[k] = (idxv + _IDX_CONST).astype(jnp.int32)


@functools.partial(jax.jit, static_argnames=("interpret",))
def kernel(x, W_in, b_in, W_out, b_out, interpret=False):
    B, T, D = x.shape

    idx3, out3 = pl.pallas_call(
        _fsq_kernel,
        grid=(B // _BB,),
        in_specs=[
            pl.BlockSpec((_BB, T, D), lambda i: (i, 0, 0)),
            pl.BlockSpec((D, _C), lambda i: (0, 0)),
            pl.BlockSpec((_C, 1), lambda i: (0, 0)),
            pl.BlockSpec((_C, 8), lambda i: (0, 0)),
            pl.BlockSpec((_C, D), lambda i: (0, 0)),
            pl.BlockSpec((D,), lambda i: (0,)),
        ],
        out_specs=[
            pl.BlockSpec((_BB, 1, T), lambda i: (i, 0, 0)),
            pl.BlockSpec((_BB, T, D), lambda i: (i, 0, 0)),
        ],
        out_shape=[
            jax.ShapeDtypeStruct((B, 1, T), jnp.int32),
            jax.ShapeDtypeStruct((B, T, D), jnp.float32),
        ],
        compiler_params=pltpu.CompilerParams(
            dimension_semantics=("parallel",)),
        interpret=interpret,
    )(x, W_in, b_in.reshape(_C, 1), jnp.asarray(_CVECT), W_out, b_out)

    embed_ind = idx3.reshape(B, T)
    commit_loss = jnp.zeros((), dtype=jnp.float32)
    return (embed_ind, out3, commit_loss)


# speedup vs baseline: 1.2095x; 1.1271x over previous
import functools

import jax
import jax.numpy as jnp
import numpy as np
from jax.experimental import pallas as pl
from jax.experimental.pallas import tpu as pltpu

_LEVELS = np.array([8, 8, 8, 6, 5], dtype=np.int64)
_DIM = 768
_C = len(_LEVELS)

_EPS = 1e-3
_levels_f = _LEVELS.astype(np.float32)
_half_l = ((_levels_f - 1.0) * (1.0 + _EPS) / 2.0).astype(np.float32)
_offset = np.where(_LEVELS % 2 == 0, 0.5, 0.0).astype(np.float32)
_shift = np.arctanh(_offset / _half_l).astype(np.float32)
_half_width = (_LEVELS // 2).astype(np.float32)
_basis = np.concatenate(([1], np.cumprod(_LEVELS[:-1]))).astype(np.float32)
_IDX_CONST = float(np.sum(_half_width * _basis))

_CVECT = np.zeros((_C, 8), dtype=np.float32)
_CVECT[:, 0] = _shift
_CVECT[:, 1] = _half_l
_CVECT[:, 2] = _offset
_CVECT[:, 3] = _basis
_CVECT[:, 4] = 1.0 / _half_width

_BB = 4


def _fsq_kernel(x_ref, w_int_ref, b_in_ref, cvec_ref, w_out_ref, b_out_ref,
                idx_ref, out_ref):
    bb, t, d = x_ref.shape
    x = x_ref[...].reshape(bb * t, d)
    zt = jax.lax.dot_general(w_int_ref[...], x, (((1,), (1,)), ((), ())),
                             preferred_element_type=jnp.float32)
    shift_eff = b_in_ref[...].reshape(_C, 1) + cvec_ref[:, 0:1]
    bounded = (jnp.tanh(zt + shift_eff) * cvec_ref[:, 1:2]
               - cvec_ref[:, 2:3])
    qt = jnp.round(bounded)
    codes_t = qt * cvec_ref[:, 4:5]
    out = jax.lax.dot_general(codes_t, w_out_ref[...],
                              (((0,), (0,)), ((), ())),
                              preferred_element_type=jnp.float32)
    out_ref[...] = (out + b_out_ref[...][None, :]).reshape(bb, t, d)
    half = (pl.program_id(0) % 2) * bb
    for k in range(bb):
        idxv = jax.lax.dot_general(cvec_ref[:, 3:4], qt[:, k * t:(k + 1) * t],
                                   (((0,), (0,)), ((), ())),
                                   preferred_element_type=jnp.float32)
        idx_ref[pl.ds(half + k, 1), :] = (idxv + _IDX_CONST).astype(jnp.int32)


@functools.partial(jax.jit, static_argnames=("interpret",))
def kernel(x, W_in, b_in, W_out, b_out, interpret=False):
    B, T, D = x.shape

    idx2, out3 = pl.pallas_call(
        _fsq_kernel,
        grid=(B // _BB,),
        in_specs=[
            pl.BlockSpec((_BB, T, D), lambda i: (i, 0, 0)),
            pl.BlockSpec((_C, D), lambda i: (0, 0)),
            pl.BlockSpec((_C,), lambda i: (0,)),
            pl.BlockSpec((_C, 8), lambda i: (0, 0)),
            pl.BlockSpec((_C, D), lambda i: (0, 0)),
            pl.BlockSpec((D,), lambda i: (0,)),
        ],
        out_specs=[
            pl.BlockSpec((2 * _BB, T), lambda i: (i // 2, 0)),
            pl.BlockSpec((_BB, T, D), lambda i: (i, 0, 0)),
        ],
        out_shape=[
            jax.ShapeDtypeStruct((B, T), jnp.int32),
            jax.ShapeDtypeStruct((B, T, D), jnp.float32),
        ],
        compiler_params=pltpu.CompilerParams(
            dimension_semantics=("arbitrary",)),
        interpret=interpret,
    )(x, W_in.T, b_in, jnp.asarray(_CVECT), W_out, b_out)

    commit_loss = jnp.zeros((), dtype=jnp.float32)
    return (idx2, out3, commit_loss)
